# baseline (speedup 1.0000x reference)
import jax
import jax.numpy as jnp
from jax import lax
from jax.experimental import pallas as pl
from jax.experimental.pallas import tpu as pltpu
from jax.experimental.pallas import tpu_sc as plsc

N = 65536
S = 128
NF = 64
NW = 32
RAYS_PER_W = N // NW
GROUPS = RAYS_PER_W // 16


def _sc_body(w_hbm, t_hbm, o_hbm, d_hbm, pts_hbm, ft_hbm,
             wv2, tv2, ov2, dv2, sraw, kmap, ptsv, ftv, sem2):
    c = lax.axis_index("c")
    sax = lax.axis_index("s")
    wid = sax * 2 + c
    iota = lax.iota(jnp.int32, 16)
    zeros_i = jnp.zeros((16,), jnp.int32)
    zeros_f = jnp.zeros((16,), jnp.float32)
    one16 = jnp.full((16,), 1, jnp.int32)
    two16 = jnp.full((16,), 2, jnp.int32)

    def issue(g, b):
        base = (wid * GROUPS + g) * 16
        pltpu.async_copy(w_hbm.at[pl.ds(base, 16)], wv2.at[b], sem2.at[b])
        pltpu.async_copy(t_hbm.at[pl.ds(base, 16)], tv2.at[b], sem2.at[b])
        pltpu.async_copy(o_hbm.at[pl.ds(base, 16)], ov2.at[b], sem2.at[b])
        pltpu.async_copy(d_hbm.at[pl.ds(base, 16)], dv2.at[b], sem2.at[b])

    def drain(g, b):
        base = (wid * GROUPS + g) * 16
        pltpu.make_async_copy(w_hbm.at[pl.ds(base, 16)], wv2.at[b],
                              sem2.at[b]).wait()
        pltpu.make_async_copy(t_hbm.at[pl.ds(base, 16)], tv2.at[b],
                              sem2.at[b]).wait()
        pltpu.make_async_copy(o_hbm.at[pl.ds(base, 16)], ov2.at[b],
                              sem2.at[b]).wait()
        pltpu.make_async_copy(d_hbm.at[pl.ds(base, 16)], dv2.at[b],
                              sem2.at[b]).wait()

    def compute(g, b):
        base = (wid * GROUPS + g) * 16
        wv, tv, ov, dv = wv2.at[b], tv2.at[b], ov2.at[b], dv2.at[b]

        for j in range(NF):
            kmap[j] = zeros_i

        def p1(s, run):
            wcol = plsc.load_gather(wv, [iota, jnp.full((16,), s, jnp.int32)])
            run = run + wcol + jnp.float32(1e-5)
            sraw[s] = run
            return run
        tot = lax.fori_loop(0, S, p1, zeros_f, unroll=8)
        inv_t = jnp.float32(1.0) / tot
        scale = inv_t * jnp.float32(NF - 1)

        def p2(s, jfm1):
            x = sraw[s] * scale
            ti = x.astype(jnp.int32)
            jf = jnp.where(x > ti.astype(jnp.float32), ti + 1, ti)
            jf = jnp.minimum(jf, jnp.int32(NF))
            mask = jf > jfm1
            plsc.store_scatter(kmap,
                               [jnp.minimum(jfm1, jnp.int32(NF - 1)), iota],
                               jnp.full((16,), s, jnp.int32), mask=mask)
            return jf
        jlast = lax.fori_loop(0, S, p2, zeros_i, unroll=8)
        tmask = jlast <= jnp.int32(NF - 1)
        plsc.store_scatter(kmap,
                           [jnp.minimum(jlast, jnp.int32(NF - 1)), iota],
                           jnp.full((16,), S, jnp.int32), mask=tmask)

        def pf(j, runk):
            runk = jnp.maximum(runk, kmap[j])
            kmap[j] = runk
            return runk
        lax.fori_loop(0, NF, pf, zeros_i, unroll=8)

        ox = plsc.load_gather(ov, [iota, zeros_i])
        oy = plsc.load_gather(ov, [iota, one16])
        oz = plsc.load_gather(ov, [iota, two16])
        dx = plsc.load_gather(dv, [iota, zeros_i])
        dy = plsc.load_gather(dv, [iota, one16])
        dz = plsc.load_gather(dv, [iota, two16])

        def pj(j, _):
            jvec = jnp.full((16,), j, jnp.int32)
            k = kmap[j]
            km1 = jnp.maximum(k - 1, 0)
            srm1 = plsc.load_gather(sraw, [km1, iota])
            cdf_b = jnp.where(k == 0, jnp.float32(0.0), srm1 * inv_t)
            k2 = jnp.minimum(k, jnp.int32(S - 1))
            cdf_a = plsc.load_gather(sraw, [k2, iota]) * inv_t
            t_b = plsc.load_gather(tv, [iota, k2])
            k3 = jnp.minimum(k + 1, jnp.int32(S - 1))
            t_a = plsc.load_gather(tv, [iota, k3])
            denom = cdf_a - cdf_b
            denom = jnp.where(denom < jnp.float32(1e-5), jnp.float32(1.0),
                              denom)
            uv = jnp.where(jvec == jnp.int32(NF - 1), jnp.float32(1.0),
                           jvec.astype(jnp.float32)
                           * jnp.float32(1.0 / (NF - 1)))
            ftj = t_b + (uv - cdf_b) / denom * (t_a - t_b)
            plsc.store_scatter(ftv, [iota, jvec], ftj)
            plsc.store_scatter(ptsv, [iota, jvec, zeros_i], ox + dx * ftj)
            plsc.store_scatter(ptsv, [iota, jvec, one16], oy + dy * ftj)
            plsc.store_scatter(ptsv, [iota, jvec, two16], oz + dz * ftj)
            return 0
        lax.fori_loop(0, NF, pj, 0, unroll=4)

        pltpu.sync_copy(ftv, ft_hbm.at[pl.ds(base, 16)])
        pltpu.sync_copy(ptsv, pts_hbm.at[pl.ds(base, 16)])

    issue(0, 0)
    issue(1, 1)

    def body2(h, carry):
        g0 = 2 * h
        g1 = 2 * h + 1
        drain(g0, 0)
        compute(g0, 0)
        issue(lax.rem(g0 + 2, GROUPS), 0)
        drain(g1, 1)
        compute(g1, 1)
        issue(lax.rem(g1 + 2, GROUPS), 1)
        return carry

    lax.fori_loop(0, GROUPS // 2, body2, 0)
    drain(0, 0)
    drain(1, 1)


def kernel(ray_origins, ray_directions, coarse_weights, coarse_t_vals,
           num_fine_samples):
    del num_fine_samples
    sc_call = pl.kernel(
        _sc_body,
        out_type=[
            jax.ShapeDtypeStruct((N, NF, 3), jnp.float32),
            jax.ShapeDtypeStruct((N, NF), jnp.float32),
        ],
        mesh=plsc.VectorSubcoreMesh(core_axis_name="c", subcore_axis_name="s"),
        compiler_params=pltpu.CompilerParams(needs_layout_passes=False,
                                             use_tc_tiling_on_sc=False),
        scratch_types=[
            pltpu.VMEM((2, 16, S), jnp.float32),
            pltpu.VMEM((2, 16, S), jnp.float32),
            pltpu.VMEM((2, 16, 3), jnp.float32),
            pltpu.VMEM((2, 16, 3), jnp.float32),
            pltpu.VMEM((S, 16), jnp.float32),
            pltpu.VMEM((NF, 16), jnp.int32),
            pltpu.VMEM((16, NF, 3), jnp.float32),
            pltpu.VMEM((16, NF), jnp.float32),
            pltpu.SemaphoreType.DMA((2,)),
        ],
    )
    pts, ft = sc_call(coarse_weights, coarse_t_vals, ray_origins,
                      ray_directions)
    return (pts, ft)


# speedup vs baseline: 2.5563x; 2.5563x over previous
import jax
import jax.numpy as jnp
from jax import lax
from jax.experimental import pallas as pl
from jax.experimental.pallas import tpu as pltpu
from jax.experimental.pallas import tpu_sc as plsc

N = 65536
S = 128
NF = 64
NW = 32
RAYS_PER_W = N // NW
GROUPS = RAYS_PER_W // 16


def _sc_body(wt_hbm, t_hbm, o_hbm, d_hbm, pts_hbm, ft_hbm,
             wtv2, tv2, ov2, dv2, sraw, kmap, ptsv, ftv, sem2):
    c = lax.axis_index("c")
    sax = lax.axis_index("s")
    wid = sax * 2 + c
    iota = lax.iota(jnp.int32, 16)
    zeros_i = jnp.zeros((16,), jnp.int32)
    zeros_f = jnp.zeros((16,), jnp.float32)
    one16 = jnp.full((16,), 1, jnp.int32)
    two16 = jnp.full((16,), 2, jnp.int32)

    def issue(g, b):
        base = (wid * GROUPS + g) * 16
        pltpu.async_copy(wt_hbm.at[pl.ds(0, S), pl.ds(base, 16)],
                         wtv2.at[b], sem2.at[b])
        pltpu.async_copy(t_hbm.at[pl.ds(base, 16)], tv2.at[b], sem2.at[b])
        pltpu.async_copy(o_hbm.at[pl.ds(base, 16)], ov2.at[b], sem2.at[b])
        pltpu.async_copy(d_hbm.at[pl.ds(base, 16)], dv2.at[b], sem2.at[b])

    def drain(g, b):
        base = (wid * GROUPS + g) * 16
        pltpu.make_async_copy(wt_hbm.at[pl.ds(0, S), pl.ds(base, 16)],
                              wtv2.at[b], sem2.at[b]).wait()
        pltpu.make_async_copy(t_hbm.at[pl.ds(base, 16)], tv2.at[b],
                              sem2.at[b]).wait()
        pltpu.make_async_copy(o_hbm.at[pl.ds(base, 16)], ov2.at[b],
                              sem2.at[b]).wait()
        pltpu.make_async_copy(d_hbm.at[pl.ds(base, 16)], dv2.at[b],
                              sem2.at[b]).wait()

    def compute(g, b):
        base = (wid * GROUPS + g) * 16
        wtv, tv, ov, dv = wtv2.at[b], tv2.at[b], ov2.at[b], dv2.at[b]

        for j in range(NF):
            kmap[j] = zeros_i

        def p1(s, run):
            run = run + wtv[s] + jnp.float32(1e-5)
            sraw[s] = run
            return run
        tot = lax.fori_loop(0, S, p1, zeros_f, unroll=8)
        inv_t = jnp.float32(1.0) / tot
        scale = inv_t * jnp.float32(NF - 1)

        def p2(s, jfm1):
            x = sraw[s] * scale
            ti = x.astype(jnp.int32)
            jf = jnp.where(x > ti.astype(jnp.float32), ti + 1, ti)
            jf = jnp.minimum(jf, jnp.int32(NF))
            mask = jf > jfm1
            plsc.store_scatter(kmap,
                               [jnp.minimum(jfm1, jnp.int32(NF - 1)), iota],
                               jnp.full((16,), s, jnp.int32), mask=mask)
            return jf
        jlast = lax.fori_loop(0, S, p2, zeros_i, unroll=8)
        tmask = jlast <= jnp.int32(NF - 1)
        plsc.store_scatter(kmap,
                           [jnp.minimum(jlast, jnp.int32(NF - 1)), iota],
                           jnp.full((16,), S, jnp.int32), mask=tmask)

        def pf(j, runk):
            runk = jnp.maximum(runk, kmap[j])
            kmap[j] = runk
            return runk
        lax.fori_loop(0, NF, pf, zeros_i, unroll=8)

        ox = plsc.load_gather(ov, [iota, zeros_i])
        oy = plsc.load_gather(ov, [iota, one16])
        oz = plsc.load_gather(ov, [iota, two16])
        dx = plsc.load_gather(dv, [iota, zeros_i])
        dy = plsc.load_gather(dv, [iota, one16])
        dz = plsc.load_gather(dv, [iota, two16])

        def pj(j, _):
            jvec = jnp.full((16,), j, jnp.int32)
            k = kmap[j]
            km1 = jnp.maximum(k - 1, 0)
            srm1 = plsc.load_gather(sraw, [km1, iota])
            cdf_b = jnp.where(k == 0, jnp.float32(0.0), srm1 * inv_t)
            k2 = jnp.minimum(k, jnp.int32(S - 1))
            cdf_a = plsc.load_gather(sraw, [k2, iota]) * inv_t
            t_b = plsc.load_gather(tv, [iota, k2])
            k3 = jnp.minimum(k + 1, jnp.int32(S - 1))
            t_a = plsc.load_gather(tv, [iota, k3])
            denom = cdf_a - cdf_b
            denom = jnp.where(denom < jnp.float32(1e-5), jnp.float32(1.0),
                              denom)
            uv = jnp.where(jvec == jnp.int32(NF - 1), jnp.float32(1.0),
                           jvec.astype(jnp.float32)
                           * jnp.float32(1.0 / (NF - 1)))
            ftj = t_b + (uv - cdf_b) / denom * (t_a - t_b)
            plsc.store_scatter(ftv, [iota, jvec], ftj)
            plsc.store_scatter(ptsv, [zeros_i, iota, jvec], ox + dx * ftj)
            plsc.store_scatter(ptsv, [one16, iota, jvec], oy + dy * ftj)
            plsc.store_scatter(ptsv, [two16, iota, jvec], oz + dz * ftj)
            return 0
        lax.fori_loop(0, NF, pj, 0, unroll=4)

        pltpu.sync_copy(ftv, ft_hbm.at[pl.ds(base, 16)])
        pltpu.sync_copy(ptsv, pts_hbm.at[pl.ds(0, 3), pl.ds(base, 16)])

    issue(0, 0)
    issue(1, 1)

    def body2(h, carry):
        g0 = 2 * h
        g1 = 2 * h + 1
        drain(g0, 0)
        compute(g0, 0)
        issue(lax.rem(g0 + 2, GROUPS), 0)
        drain(g1, 1)
        compute(g1, 1)
        issue(lax.rem(g1 + 2, GROUPS), 1)
        return carry

    lax.fori_loop(0, GROUPS // 2, body2, 0)
    drain(0, 0)
    drain(1, 1)


def kernel(ray_origins, ray_directions, coarse_weights, coarse_t_vals,
           num_fine_samples):
    del num_fine_samples
    sc_call = pl.kernel(
        _sc_body,
        out_type=[
            jax.ShapeDtypeStruct((3, N, NF), jnp.float32),
            jax.ShapeDtypeStruct((N, NF), jnp.float32),
        ],
        mesh=plsc.VectorSubcoreMesh(core_axis_name="c", subcore_axis_name="s"),
        compiler_params=pltpu.CompilerParams(needs_layout_passes=False,
                                             use_tc_tiling_on_sc=False),
        scratch_types=[
            pltpu.VMEM((2, S, 16), jnp.float32),
            pltpu.VMEM((2, 16, S), jnp.float32),
            pltpu.VMEM((2, 16, 3), jnp.float32),
            pltpu.VMEM((2, 16, 3), jnp.float32),
            pltpu.VMEM((S, 16), jnp.float32),
            pltpu.VMEM((NF, 16), jnp.int32),
            pltpu.VMEM((3, 16, NF), jnp.float32),
            pltpu.VMEM((16, NF), jnp.float32),
            pltpu.SemaphoreType.DMA((2,)),
        ],
    )
    w_t = coarse_weights.T
    pts, ft = sc_call(w_t, coarse_t_vals, ray_origins, ray_directions)
    fine_points = jnp.transpose(pts, (1, 2, 0))
    return (fine_points, ft)
